# baseline (speedup 1.0000x reference)
import jax
import jax.numpy as jnp
from jax.experimental import pallas as pl
from jax.experimental.pallas import tpu as pltpu

_BM = 512


def _gate_kernel(x_ref, wt_ref, out_ref):
    logits = jnp.dot(x_ref[...], wt_ref[...], preferred_element_type=jnp.float32)
    m = jnp.max(logits, axis=1, keepdims=True)
    e = jnp.exp(logits - m)
    out_ref[...] = e / jnp.sum(e, axis=1, keepdims=True)


def kernel(inputs, W):
    tokens, d = inputs.shape
    n_exp = W.shape[0]
    wt = W.T
    return pl.pallas_call(
        _gate_kernel,
        grid=(tokens // _BM,),
        in_specs=[
            pl.BlockSpec((_BM, d), lambda i: (i, 0)),
            pl.BlockSpec((d, n_exp), lambda i: (0, 0)),
        ],
        out_specs=pl.BlockSpec((_BM, n_exp), lambda i: (i, 0)),
        out_shape=jax.ShapeDtypeStruct((tokens, n_exp), jnp.float32),
        compiler_params=pltpu.CompilerParams(
            dimension_semantics=("arbitrary",),
        ),
    )(inputs, wt)


# speedup vs baseline: 1.0001x; 1.0001x over previous
import jax
import jax.numpy as jnp
from jax.experimental import pallas as pl
from jax.experimental.pallas import tpu as pltpu

_BM = 512


def _gate_kernel(x_ref, wt_ref, out_ref):
    logits = jnp.dot(x_ref[...], wt_ref[...], preferred_element_type=jnp.float32,
                     precision=jax.lax.Precision.DEFAULT)
    m = jnp.max(logits, axis=1, keepdims=True)
    e = jnp.exp(logits - m)
    out_ref[...] = e / jnp.sum(e, axis=1, keepdims=True)


def kernel(inputs, W):
    tokens, d = inputs.shape
    n_exp = W.shape[0]
    wt = W.T
    return pl.pallas_call(
        _gate_kernel,
        grid=(tokens // _BM,),
        in_specs=[
            pl.BlockSpec((_BM, d), lambda i: (i, 0)),
            pl.BlockSpec((d, n_exp), lambda i: (0, 0)),
        ],
        out_specs=pl.BlockSpec((_BM, n_exp), lambda i: (i, 0)),
        out_shape=jax.ShapeDtypeStruct((tokens, n_exp), jnp.float32),
        compiler_params=pltpu.CompilerParams(
            dimension_semantics=("arbitrary",),
        ),
    )(inputs, wt)


# speedup vs baseline: 1.0103x; 1.0102x over previous
import jax
import jax.numpy as jnp
from jax.experimental import pallas as pl
from jax.experimental.pallas import tpu as pltpu

_BM = 1024


def _gate_kernel(x_ref, wt_ref, out_ref):
    logits = jnp.dot(x_ref[...], wt_ref[...], preferred_element_type=jnp.float32,
                     precision=jax.lax.Precision.DEFAULT)
    m = jnp.max(logits, axis=1, keepdims=True)
    e = jnp.exp(logits - m)
    out_ref[...] = e / jnp.sum(e, axis=1, keepdims=True)


def kernel(inputs, W):
    tokens, d = inputs.shape
    n_exp = W.shape[0]
    wt = W.T
    return pl.pallas_call(
        _gate_kernel,
        grid=(tokens // _BM,),
        in_specs=[
            pl.BlockSpec((_BM, d), lambda i: (i, 0)),
            pl.BlockSpec((d, n_exp), lambda i: (0, 0)),
        ],
        out_specs=pl.BlockSpec((_BM, n_exp), lambda i: (i, 0)),
        out_shape=jax.ShapeDtypeStruct((tokens, n_exp), jnp.float32),
        compiler_params=pltpu.CompilerParams(
            dimension_semantics=("arbitrary",),
        ),
    )(inputs, wt)
